# Initial kernel scaffold; baseline (speedup 1.0000x reference)
import jax
import jax.numpy as jnp
from jax.experimental import pallas as pl


def kernel(x, edge_index, attr, W_l, b_l, W_r):
    raise NotImplementedError("write your pallas kernel here")


# speedup vs baseline: 8.5059x; 8.5059x over previous
import functools

import jax
import jax.numpy as jnp
from jax import lax
from jax.experimental import pallas as pl
from jax.experimental.pallas import tpu as pltpu
from jax.experimental.pallas import tpu_sc as plsc

N = 10000
E = 320000
D = 128
NC = 2
NS = 16
NW = NC * NS
EPW = E // NW
C = 200
CHUNKS = EPW // C
RPT = 624
ZR = 104
TAIL = N - NS * RPT
CNT_B = 2000


def _sc_aggregate(x, src, dst):
  mesh = plsc.VectorSubcoreMesh(core_axis_name="c", subcore_axis_name="s")

  @functools.partial(
      pl.kernel,
      out_type=(
          jax.ShapeDtypeStruct((NC, N, D), jnp.float32),
          jax.ShapeDtypeStruct((NC * N,), jnp.float32),
      ),
      mesh=mesh,
      scratch_types=[
          pltpu.VMEM((C, D), jnp.float32),
          pltpu.VMEM((C,), jnp.int32),
          pltpu.VMEM((C,), jnp.int32),
          pltpu.VMEM((C,), jnp.float32),
          pltpu.VMEM((CNT_B,), jnp.float32),
          pltpu.VMEM_SHARED((N, D), jnp.float32),
          pltpu.VMEM_SHARED((N,), jnp.float32),
          pltpu.SemaphoreType.DMA,
      ],
  )
  def agg(x_hbm, src_hbm, dst_hbm, acc_out, cnt_out,
          rows_v, sidx_v, didx_v, ones_v, zc_v, acc_sh, cnt_sh, sem):
    cid = lax.axis_index("c")
    tid = lax.axis_index("s")
    wid = cid * NS + tid
    zeros16 = jnp.zeros((16,), jnp.float32)
    ones16 = jnp.ones((16,), jnp.float32)

    def zrow(r, carry):
      for g in range(D // 16):
        rows_v[r, pl.ds(g * 16, 16)] = zeros16
      return carry
    lax.fori_loop(0, ZR, zrow, 0)

    for j in range(RPT // ZR):
      pltpu.sync_copy(rows_v.at[pl.ds(0, ZR)],
                      acc_sh.at[pl.ds(tid * RPT + j * ZR, ZR)])

    @pl.when(tid == 0)
    def _():
      pltpu.sync_copy(rows_v.at[pl.ds(0, TAIL)],
                      acc_sh.at[pl.ds(NS * RPT, TAIL)])

    def zcnt(k, carry):
      zc_v[pl.ds(k * 16, 16)] = zeros16
      return carry
    lax.fori_loop(0, CNT_B // 16, zcnt, 0)

    def fones(k, carry):
      ones_v[pl.ds(k * 16, 16)] = ones16
      return carry
    lax.fori_loop(0, C // 16, fones, 0)

    @pl.when(tid < N // CNT_B)
    def _():
      pltpu.sync_copy(zc_v, cnt_sh.at[pl.ds(tid * CNT_B, CNT_B)])

    plsc.subcore_barrier()

    def chunk(i, carry):
      base = pl.multiple_of(wid * EPW + i * C, 8)
      pltpu.sync_copy(src_hbm.at[pl.ds(base, C)], sidx_v)
      pltpu.sync_copy(dst_hbm.at[pl.ds(base, C)], didx_v)
      pltpu.async_copy(x_hbm.at[sidx_v], rows_v, sem).wait()
      pltpu.sync_copy(rows_v, acc_sh.at[didx_v], add=True)
      pltpu.sync_copy(ones_v, cnt_sh.at[didx_v], add=True)
      return carry
    lax.fori_loop(0, CHUNKS, chunk, 0)

    plsc.subcore_barrier()

    for j in range(RPT // ZR):
      r0 = tid * RPT + j * ZR
      pltpu.sync_copy(acc_sh.at[pl.ds(r0, ZR)], rows_v.at[pl.ds(0, ZR)])
      pltpu.sync_copy(rows_v.at[pl.ds(0, ZR)], acc_out.at[cid, pl.ds(r0, ZR)])

    @pl.when(tid == 0)
    def _():
      pltpu.sync_copy(acc_sh.at[pl.ds(NS * RPT, TAIL)], rows_v.at[pl.ds(0, TAIL)])
      pltpu.sync_copy(rows_v.at[pl.ds(0, TAIL)],
                      acc_out.at[cid, pl.ds(NS * RPT, TAIL)])

    @pl.when(tid < N // CNT_B)
    def _():
      pltpu.sync_copy(cnt_sh.at[pl.ds(tid * CNT_B, CNT_B)], zc_v)
      pltpu.sync_copy(zc_v, cnt_out.at[pl.ds(cid * N + tid * CNT_B, CNT_B)])

  parts, cnt_flat = agg(x, src, dst)
  return parts, cnt_flat


def _tc_finish(parts, cnts3, x, wlt, b2, wrt):
  def body(parts_ref, cnts_ref, x_ref, wl_ref, b_ref, wr_ref, o_ref):
    sums = parts_ref[0] + parts_ref[1]
    cnt = cnts_ref[0] + cnts_ref[1]
    mean = sums / jnp.maximum(cnt, 1.0)
    out = (jnp.dot(mean, wl_ref[...], preferred_element_type=jnp.float32,
                   precision=lax.Precision.HIGHEST)
           + jnp.dot(x_ref[...], wr_ref[...], preferred_element_type=jnp.float32,
                     precision=lax.Precision.HIGHEST)
           + b_ref[...])
    nrm = jnp.sqrt(jnp.sum(out * out, axis=1, keepdims=True))
    o_ref[...] = out / jnp.maximum(nrm, 1e-12)

  BN = 2000
  return pl.pallas_call(
      body,
      grid=(N // BN,),
      in_specs=[
          pl.BlockSpec((NC, BN, D), lambda i: (0, i, 0)),
          pl.BlockSpec((NC, BN, 1), lambda i: (0, i, 0)),
          pl.BlockSpec((BN, D), lambda i: (i, 0)),
          pl.BlockSpec((D, D), lambda i: (0, 0)),
          pl.BlockSpec((1, D), lambda i: (0, 0)),
          pl.BlockSpec((D, D), lambda i: (0, 0)),
      ],
      out_specs=pl.BlockSpec((BN, D), lambda i: (i, 0)),
      out_shape=jax.ShapeDtypeStruct((N, D), jnp.float32),
  )(parts, cnts3, x, wlt, b2, wrt)


def kernel(x, edge_index, attr, W_l, b_l, W_r):
  src = edge_index[0]
  dst = edge_index[1]
  parts, cnts = _sc_aggregate(x, src, dst)
  out = _tc_finish(parts, cnts.reshape(NC, N, 1), x,
                   W_l.T, b_l.reshape(1, D), W_r.T)
  return (out, edge_index, attr)


# speedup vs baseline: 12.6402x; 1.4860x over previous
import functools

import jax
import jax.numpy as jnp
from jax import lax
from jax.experimental import pallas as pl
from jax.experimental.pallas import tpu as pltpu
from jax.experimental.pallas import tpu_sc as plsc

N = 10000
E = 320000
D = 128
NC = 2
NS = 16
NW = NC * NS
EPW = E // NW
C = 176
CHUNKS = 56
ETAIL = EPW - CHUNKS * C
CB = 1000
RPT = 624
ZR = 104
TAIL = N - NS * RPT
CNT_B = 1000


def _sc_aggregate(x, src, dst):
  mesh = plsc.VectorSubcoreMesh(core_axis_name="c", subcore_axis_name="s")

  @functools.partial(
      pl.kernel,
      out_type=(
          jax.ShapeDtypeStruct((NC, N, D), jnp.float32),
          jax.ShapeDtypeStruct((NC * N,), jnp.float32),
      ),
      mesh=mesh,
      scratch_types=[
          pltpu.VMEM((C, D), jnp.float32),
          pltpu.VMEM((C, D), jnp.float32),
          pltpu.VMEM((C,), jnp.int32),
          pltpu.VMEM((C,), jnp.int32),
          pltpu.VMEM((C,), jnp.int32),
          pltpu.VMEM((C,), jnp.int32),
          pltpu.VMEM((C,), jnp.int32),
          pltpu.VMEM((C,), jnp.int32),
          pltpu.VMEM((C,), jnp.int32),
          pltpu.VMEM((C,), jnp.int32),
          pltpu.VMEM((ETAIL,), jnp.int32),
          pltpu.VMEM((ETAIL,), jnp.int32),
          pltpu.VMEM((CB,), jnp.int32),
          pltpu.VMEM((CB,), jnp.float32),
          pltpu.VMEM_SHARED((N, D), jnp.float32),
          pltpu.VMEM_SHARED((N,), jnp.float32),
          pltpu.SemaphoreType.DMA,
          pltpu.SemaphoreType.DMA,
          pltpu.SemaphoreType.DMA,
          pltpu.SemaphoreType.DMA,
          pltpu.SemaphoreType.DMA,
          pltpu.SemaphoreType.DMA,
      ],
  )
  def agg(x_hbm, src_hbm, dst_hbm, acc_out, cnt_out,
          rows0_v, rows1_v, sx0, sx1, sx2, sx3, dx0, dx1, dx2, dx3,
          stail_v, dtail_v, cidx_v, zo_v,
          acc_sh, cnt_sh, sg0, sg1, si0, si1, si2, si3):
    cid = lax.axis_index("c")
    tid = lax.axis_index("s")
    wid = cid * NS + tid
    ebase = wid * EPW
    zeros16 = jnp.zeros((16,), jnp.float32)
    ones16 = jnp.ones((16,), jnp.float32)
    rows = (rows0_v, rows1_v)
    sidx = (sx0, sx1, sx2, sx3)
    didx = (dx0, dx1, dx2, dx3)
    sgs = (sg0, sg1)
    sis = (si0, si1, si2, si3)

    def zrow(r, carry):
      for g in range(D // 16):
        rows0_v[r, pl.ds(g * 16, 16)] = zeros16
      return carry
    lax.fori_loop(0, ZR, zrow, 0)

    def zcnt(k, carry):
      zo_v[pl.ds(k * 16, 16)] = zeros16
      return carry
    lax.fori_loop(0, CB // 16, zcnt, 0)
    zo_v[pl.ds(CB - 16, 16)] = zeros16

    for j in range(RPT // ZR):
      pltpu.sync_copy(rows0_v.at[pl.ds(0, ZR)],
                      acc_sh.at[pl.ds(tid * RPT + j * ZR, ZR)])

    @pl.when(tid == 0)
    def _():
      pltpu.sync_copy(rows0_v.at[pl.ds(0, TAIL)],
                      acc_sh.at[pl.ds(NS * RPT, TAIL)])

    @pl.when(tid < N // CNT_B)
    def _():
      pltpu.sync_copy(zo_v.at[pl.ds(0, CNT_B)], cnt_sh.at[pl.ds(tid * CNT_B, CNT_B)])

    plsc.subcore_barrier()

    def idx_start(g, q):
      base = pl.multiple_of(ebase + g * C, 8)
      pltpu.make_async_copy(src_hbm.at[pl.ds(base, C)], sidx[q], sis[q]).start()
      pltpu.make_async_copy(dst_hbm.at[pl.ds(base, C)], didx[q], sis[q]).start()

    def idx_wait(q):
      pltpu.make_async_copy(src_hbm.at[pl.ds(0, C)], sidx[q], sis[q]).wait()
      pltpu.make_async_copy(dst_hbm.at[pl.ds(0, C)], didx[q], sis[q]).wait()

    def gather_start(q, b):
      pltpu.make_async_copy(x_hbm.at[sidx[q]], rows[b], sgs[b]).start()

    def gather_wait(q, b):
      pltpu.make_async_copy(x_hbm.at[sidx[q]], rows[b], sgs[b]).wait()

    idx_start(0, 0)
    idx_start(1, 1)
    idx_wait(0)
    gather_start(0, 0)
    idx_wait(1)
    gather_start(1, 1)
    idx_start(2, 2)
    idx_start(3, 3)

    def quad(k4, carry):
      g0 = 4 * k4
      for j in range(4):
        g = g0 + j
        q = j
        b = j % 2
        gather_wait(q, b)
        pltpu.sync_copy(rows[b], acc_sh.at[didx[q]], add=True)
        idx_wait((q + 2) % 4)
        gather_start((q + 2) % 4, b)
        idx_start(g + 4, q)
      return carry
    NQ = (CHUNKS - 8) // 4 + 1
    lax.fori_loop(0, NQ, quad, 0)

    for j in range(4):
      g = NQ * 4 + j
      q = j
      b = j % 2
      gather_wait(q, b)
      pltpu.sync_copy(rows[b], acc_sh.at[didx[q]], add=True)
      if j < 2:
        idx_wait((q + 2) % 4)
        gather_start((q + 2) % 4, b)

    tbase = pl.multiple_of(ebase + CHUNKS * C, 8)
    pltpu.sync_copy(src_hbm.at[pl.ds(tbase, ETAIL)], stail_v)
    pltpu.sync_copy(dst_hbm.at[pl.ds(tbase, ETAIL)], dtail_v)
    pltpu.make_async_copy(x_hbm.at[stail_v], rows0_v.at[pl.ds(0, ETAIL)], sg0).start()
    pltpu.make_async_copy(x_hbm.at[stail_v], rows0_v.at[pl.ds(0, ETAIL)], sg0).wait()
    pltpu.sync_copy(rows0_v.at[pl.ds(0, ETAIL)], acc_sh.at[dtail_v], add=True)

    def fones(k, carry):
      zo_v[pl.ds(k * 16, 16)] = ones16
      return carry
    lax.fori_loop(0, CB // 16, fones, 0)
    zo_v[pl.ds(CB - 16, 16)] = ones16

    def cchunk(i, carry):
      base = pl.multiple_of(ebase + i * CB, 8)
      pltpu.sync_copy(dst_hbm.at[pl.ds(base, CB)], cidx_v)
      pltpu.sync_copy(zo_v, cnt_sh.at[cidx_v], add=True)
      return carry
    lax.fori_loop(0, EPW // CB, cchunk, 0)

    plsc.subcore_barrier()

    for j in range(RPT // ZR):
      r0 = tid * RPT + j * ZR
      pltpu.sync_copy(acc_sh.at[pl.ds(r0, ZR)], rows0_v.at[pl.ds(0, ZR)])
      pltpu.sync_copy(rows0_v.at[pl.ds(0, ZR)], acc_out.at[cid, pl.ds(r0, ZR)])

    @pl.when(tid == 0)
    def _():
      pltpu.sync_copy(acc_sh.at[pl.ds(NS * RPT, TAIL)], rows0_v.at[pl.ds(0, TAIL)])
      pltpu.sync_copy(rows0_v.at[pl.ds(0, TAIL)],
                      acc_out.at[cid, pl.ds(NS * RPT, TAIL)])

    @pl.when(tid < N // CNT_B)
    def _():
      pltpu.sync_copy(cnt_sh.at[pl.ds(tid * CNT_B, CNT_B)], zo_v.at[pl.ds(0, CNT_B)])
      pltpu.sync_copy(zo_v.at[pl.ds(0, CNT_B)],
                      cnt_out.at[pl.ds(cid * N + tid * CNT_B, CNT_B)])

  parts, cnt_flat = agg(x, src, dst)
  return parts, cnt_flat


def _tc_finish(parts, cnts3, x, wlt, b2, wrt):
  def body(parts_ref, cnts_ref, x_ref, wl_ref, b_ref, wr_ref, o_ref):
    sums = parts_ref[0] + parts_ref[1]
    cnt = cnts_ref[0] + cnts_ref[1]
    mean = sums / jnp.maximum(cnt, 1.0)
    out = (jnp.dot(mean, wl_ref[...], preferred_element_type=jnp.float32,
                   precision=lax.Precision.HIGHEST)
           + jnp.dot(x_ref[...], wr_ref[...], preferred_element_type=jnp.float32,
                     precision=lax.Precision.HIGHEST)
           + b_ref[...])
    nrm = jnp.sqrt(jnp.sum(out * out, axis=1, keepdims=True))
    o_ref[...] = out / jnp.maximum(nrm, 1e-12)

  BN = 2000
  return pl.pallas_call(
      body,
      grid=(N // BN,),
      in_specs=[
          pl.BlockSpec((NC, BN, D), lambda i: (0, i, 0)),
          pl.BlockSpec((NC, BN, 1), lambda i: (0, i, 0)),
          pl.BlockSpec((BN, D), lambda i: (i, 0)),
          pl.BlockSpec((D, D), lambda i: (0, 0)),
          pl.BlockSpec((1, D), lambda i: (0, 0)),
          pl.BlockSpec((D, D), lambda i: (0, 0)),
      ],
      out_specs=pl.BlockSpec((BN, D), lambda i: (i, 0)),
      out_shape=jax.ShapeDtypeStruct((N, D), jnp.float32),
  )(parts, cnts3, x, wlt, b2, wrt)


def kernel(x, edge_index, attr, W_l, b_l, W_r):
  src = edge_index[0]
  dst = edge_index[1]
  parts, cnts = _sc_aggregate(x, src, dst)
  out = _tc_finish(parts, cnts.reshape(NC, N, 1), x,
                   W_l.T, b_l.reshape(1, D), W_r.T)
  return (out, edge_index, attr)


# speedup vs baseline: 13.4408x; 1.0633x over previous
import functools

import jax
import jax.numpy as jnp
from jax import lax
from jax.experimental import pallas as pl
from jax.experimental.pallas import tpu as pltpu
from jax.experimental.pallas import tpu_sc as plsc

N = 10000
E = 320000
D = 128
NC = 2
NS = 16
NW = NC * NS
EPW = E // NW
C = 176
CHUNKS = 56
ETAIL = EPW - CHUNKS * C
CB = 1000
RPT = 624
ZR = 104
TAIL = N - NS * RPT
CNT_B = 1000


def _sc_aggregate(x, src, dst):
  mesh = plsc.VectorSubcoreMesh(core_axis_name="c", subcore_axis_name="s")

  @functools.partial(
      pl.kernel,
      out_type=(
          jax.ShapeDtypeStruct((NC, N, D), jnp.float32),
          jax.ShapeDtypeStruct((NC * N,), jnp.float32),
      ),
      mesh=mesh,
      scratch_types=[
          pltpu.VMEM((C, D), jnp.float32),
          pltpu.VMEM((C, D), jnp.float32),
          pltpu.VMEM((C,), jnp.int32),
          pltpu.VMEM((C,), jnp.int32),
          pltpu.VMEM((C,), jnp.int32),
          pltpu.VMEM((C,), jnp.int32),
          pltpu.VMEM((C,), jnp.int32),
          pltpu.VMEM((C,), jnp.int32),
          pltpu.VMEM((C,), jnp.int32),
          pltpu.VMEM((C,), jnp.int32),
          pltpu.VMEM((ETAIL,), jnp.int32),
          pltpu.VMEM((ETAIL,), jnp.int32),
          pltpu.VMEM((C,), jnp.float32),
          pltpu.VMEM((ETAIL,), jnp.float32),
          pltpu.VMEM((CB,), jnp.float32),
          pltpu.VMEM_SHARED((N, D), jnp.float32),
          pltpu.VMEM_SHARED((N,), jnp.float32),
          pltpu.SemaphoreType.DMA,
          pltpu.SemaphoreType.DMA,
          pltpu.SemaphoreType.DMA,
          pltpu.SemaphoreType.DMA,
          pltpu.SemaphoreType.DMA,
          pltpu.SemaphoreType.DMA,
          pltpu.SemaphoreType.DMA,
          pltpu.SemaphoreType.DMA,
      ],
  )
  def agg(x_hbm, src_hbm, dst_hbm, acc_out, cnt_out,
          rows0_v, rows1_v, sx0, sx1, sx2, sx3, dx0, dx1, dx2, dx3,
          stail_v, dtail_v, ones_c, ones_t, zo_v,
          acc_sh, cnt_sh, sg0, sg1, si0, si1, si2, si3, ss0, ss1):
    cid = lax.axis_index("c")
    tid = lax.axis_index("s")
    wid = cid * NS + tid
    ebase = wid * EPW
    zeros16 = jnp.zeros((16,), jnp.float32)
    ones16 = jnp.ones((16,), jnp.float32)
    rows = (rows0_v, rows1_v)
    sidx = (sx0, sx1, sx2, sx3)
    didx = (dx0, dx1, dx2, dx3)
    sgs = (sg0, sg1)
    sis = (si0, si1, si2, si3)
    sss = (ss0, ss1)

    def zrow(r, carry):
      for g in range(D // 16):
        rows0_v[r, pl.ds(g * 16, 16)] = zeros16
      return carry
    lax.fori_loop(0, ZR, zrow, 0)

    def zcnt(k, carry):
      zo_v[pl.ds(k * 16, 16)] = zeros16
      return carry
    lax.fori_loop(0, CB // 16, zcnt, 0)
    zo_v[pl.ds(CB - 16, 16)] = zeros16

    for j in range(RPT // ZR):
      pltpu.make_async_copy(rows0_v.at[pl.ds(0, ZR)],
                            acc_sh.at[pl.ds(tid * RPT + j * ZR, ZR)], ss0).start()

    @pl.when(tid == 0)
    def _():
      pltpu.make_async_copy(rows0_v.at[pl.ds(0, TAIL)],
                            acc_sh.at[pl.ds(NS * RPT, TAIL)], ss1).start()

    @pl.when(tid < N // CNT_B)
    def _():
      pltpu.make_async_copy(zo_v.at[pl.ds(0, CNT_B)],
                            cnt_sh.at[pl.ds(tid * CNT_B, CNT_B)], ss1).start()

    def fones(k, carry):
      ones_c[pl.ds(k * 16, 16)] = ones16
      return carry
    lax.fori_loop(0, C // 16, fones, 0)

    def ftones(k, carry):
      ones_t[pl.ds(k * 16, 16)] = ones16
      return carry
    lax.fori_loop(0, ETAIL // 16, ftones, 0)

    for j in range(RPT // ZR):
      pltpu.make_async_copy(rows0_v.at[pl.ds(0, ZR)],
                            acc_sh.at[pl.ds(tid * RPT + j * ZR, ZR)], ss0).wait()

    @pl.when(tid == 0)
    def _():
      pltpu.make_async_copy(rows0_v.at[pl.ds(0, TAIL)],
                            acc_sh.at[pl.ds(NS * RPT, TAIL)], ss1).wait()

    @pl.when(tid < N // CNT_B)
    def _():
      pltpu.make_async_copy(zo_v.at[pl.ds(0, CNT_B)],
                            cnt_sh.at[pl.ds(tid * CNT_B, CNT_B)], ss1).wait()

    plsc.subcore_barrier()

    def idx_start(g, q):
      base = pl.multiple_of(ebase + g * C, 8)
      pltpu.make_async_copy(src_hbm.at[pl.ds(base, C)], sidx[q], sis[q]).start()
      pltpu.make_async_copy(dst_hbm.at[pl.ds(base, C)], didx[q], sis[q]).start()

    def idx_wait(q):
      pltpu.make_async_copy(src_hbm.at[pl.ds(0, C)], sidx[q], sis[q]).wait()
      pltpu.make_async_copy(dst_hbm.at[pl.ds(0, C)], didx[q], sis[q]).wait()

    def gather_start(q, b):
      pltpu.make_async_copy(x_hbm.at[sidx[q]], rows[b], sgs[b]).start()

    def gather_wait(q, b):
      pltpu.make_async_copy(x_hbm.at[sidx[q]], rows[b], sgs[b]).wait()

    def scat_start(q, b):
      pltpu.make_async_copy(rows[b], acc_sh.at[didx[q]], sss[b]).start(add=True)
      pltpu.make_async_copy(ones_c, cnt_sh.at[didx[q]], sss[b]).start(add=True)

    def scat_wait(q, b):
      pltpu.make_async_copy(rows[b], acc_sh.at[didx[q]], sss[b]).wait()
      pltpu.make_async_copy(ones_c, cnt_sh.at[didx[q]], sss[b]).wait()

    idx_start(0, 0)
    idx_start(1, 1)
    idx_wait(0)
    gather_start(0, 0)
    idx_wait(1)
    gather_start(1, 1)
    idx_start(2, 2)
    idx_start(3, 3)

    def quad(k4, carry):
      g0 = 4 * k4
      for j in range(4):
        g = g0 + j
        q = j
        b = j % 2
        gather_wait(q, b)
        scat_start(q, b)
        idx_wait((q + 2) % 4)
        scat_wait(q, b)
        gather_start((q + 2) % 4, b)
        idx_start(g + 4, q)
      return carry
    NQ = (CHUNKS - 8) // 4 + 1
    lax.fori_loop(0, NQ, quad, 0)

    for j in range(4):
      g = NQ * 4 + j
      q = j
      b = j % 2
      gather_wait(q, b)
      scat_start(q, b)
      if j < 2:
        idx_wait((q + 2) % 4)
        scat_wait(q, b)
        gather_start((q + 2) % 4, b)
      else:
        scat_wait(q, b)

    tbase = pl.multiple_of(ebase + CHUNKS * C, 8)
    pltpu.sync_copy(src_hbm.at[pl.ds(tbase, ETAIL)], stail_v)
    pltpu.sync_copy(dst_hbm.at[pl.ds(tbase, ETAIL)], dtail_v)
    pltpu.make_async_copy(x_hbm.at[stail_v], rows0_v.at[pl.ds(0, ETAIL)], sg0).start()
    pltpu.make_async_copy(x_hbm.at[stail_v], rows0_v.at[pl.ds(0, ETAIL)], sg0).wait()
    pltpu.sync_copy(rows0_v.at[pl.ds(0, ETAIL)], acc_sh.at[dtail_v], add=True)
    pltpu.sync_copy(ones_t, cnt_sh.at[dtail_v], add=True)

    plsc.subcore_barrier()

    for j in range(RPT // ZR):
      b = j % 2
      r0 = tid * RPT + j * ZR
      if j >= 2:
        r_prev = tid * RPT + (j - 2) * ZR
        pltpu.make_async_copy(rows[b].at[pl.ds(0, ZR)],
                              acc_out.at[cid, pl.ds(r_prev, ZR)], sgs[b]).wait()
      pltpu.sync_copy(acc_sh.at[pl.ds(r0, ZR)], rows[b].at[pl.ds(0, ZR)])
      pltpu.make_async_copy(rows[b].at[pl.ds(0, ZR)],
                            acc_out.at[cid, pl.ds(r0, ZR)], sgs[b]).start()
    for j in (RPT // ZR - 2, RPT // ZR - 1):
      b = j % 2
      r0 = tid * RPT + j * ZR
      pltpu.make_async_copy(rows[b].at[pl.ds(0, ZR)],
                            acc_out.at[cid, pl.ds(r0, ZR)], sgs[b]).wait()

    @pl.when(tid == 0)
    def _():
      pltpu.sync_copy(acc_sh.at[pl.ds(NS * RPT, TAIL)], rows0_v.at[pl.ds(0, TAIL)])
      pltpu.sync_copy(rows0_v.at[pl.ds(0, TAIL)],
                      acc_out.at[cid, pl.ds(NS * RPT, TAIL)])

    @pl.when(tid < N // CNT_B)
    def _():
      pltpu.sync_copy(cnt_sh.at[pl.ds(tid * CNT_B, CNT_B)], zo_v.at[pl.ds(0, CNT_B)])
      pltpu.sync_copy(zo_v.at[pl.ds(0, CNT_B)],
                      cnt_out.at[pl.ds(cid * N + tid * CNT_B, CNT_B)])

  parts, cnt_flat = agg(x, src, dst)
  return parts, cnt_flat


def _tc_finish(parts, cnts3, x, wlt, b2, wrt):
  def body(parts_ref, cnts_ref, x_ref, wl_ref, b_ref, wr_ref, o_ref):
    sums = parts_ref[0] + parts_ref[1]
    cnt = cnts_ref[0] + cnts_ref[1]
    mean = sums / jnp.maximum(cnt, 1.0)
    out = (jnp.dot(mean, wl_ref[...], preferred_element_type=jnp.float32,
                   precision=lax.Precision.HIGHEST)
           + jnp.dot(x_ref[...], wr_ref[...], preferred_element_type=jnp.float32,
                     precision=lax.Precision.HIGHEST)
           + b_ref[...])
    nrm = jnp.sqrt(jnp.sum(out * out, axis=1, keepdims=True))
    o_ref[...] = out / jnp.maximum(nrm, 1e-12)

  BN = 2000
  return pl.pallas_call(
      body,
      grid=(N // BN,),
      in_specs=[
          pl.BlockSpec((NC, BN, D), lambda i: (0, i, 0)),
          pl.BlockSpec((NC, BN, 1), lambda i: (0, i, 0)),
          pl.BlockSpec((BN, D), lambda i: (i, 0)),
          pl.BlockSpec((D, D), lambda i: (0, 0)),
          pl.BlockSpec((1, D), lambda i: (0, 0)),
          pl.BlockSpec((D, D), lambda i: (0, 0)),
      ],
      out_specs=pl.BlockSpec((BN, D), lambda i: (i, 0)),
      out_shape=jax.ShapeDtypeStruct((N, D), jnp.float32),
  )(parts, cnts3, x, wlt, b2, wrt)


def kernel(x, edge_index, attr, W_l, b_l, W_r):
  src = edge_index[0]
  dst = edge_index[1]
  parts, cnts = _sc_aggregate(x, src, dst)
  out = _tc_finish(parts, cnts.reshape(NC, N, 1), x,
                   W_l.T, b_l.reshape(1, D), W_r.T)
  return (out, edge_index, attr)


# speedup vs baseline: 13.4562x; 1.0011x over previous
import functools

import jax
import jax.numpy as jnp
from jax import lax
from jax.experimental import pallas as pl
from jax.experimental.pallas import tpu as pltpu
from jax.experimental.pallas import tpu_sc as plsc

N = 10000
E = 320000
D = 128
NC = 2
NS = 16
NW = NC * NS
EPW = E // NW
C = 176
CHUNKS = 56
ETAIL = EPW - CHUNKS * C
CB = 1000
RPT = 624
ZR = 104
TAIL = N - NS * RPT
CNT_B = 1000


def _sc_aggregate(x, src, dst):
  mesh = plsc.VectorSubcoreMesh(core_axis_name="c", subcore_axis_name="s")

  @functools.partial(
      pl.kernel,
      out_type=(
          jax.ShapeDtypeStruct((NC, N, D), jnp.float32),
          jax.ShapeDtypeStruct((NC * N,), jnp.float32),
      ),
      mesh=mesh,
      scratch_types=[
          pltpu.VMEM((C, D), jnp.float32),
          pltpu.VMEM((C, D), jnp.float32),
          pltpu.VMEM((C,), jnp.int32),
          pltpu.VMEM((C,), jnp.int32),
          pltpu.VMEM((C,), jnp.int32),
          pltpu.VMEM((C,), jnp.int32),
          pltpu.VMEM((C,), jnp.int32),
          pltpu.VMEM((C,), jnp.int32),
          pltpu.VMEM((C,), jnp.int32),
          pltpu.VMEM((C,), jnp.int32),
          pltpu.VMEM((ETAIL,), jnp.int32),
          pltpu.VMEM((ETAIL,), jnp.int32),
          pltpu.VMEM((C,), jnp.float32),
          pltpu.VMEM((ETAIL,), jnp.float32),
          pltpu.VMEM((CB,), jnp.float32),
          pltpu.VMEM_SHARED((N, D), jnp.float32),
          pltpu.VMEM_SHARED((N,), jnp.float32),
          pltpu.SemaphoreType.DMA,
          pltpu.SemaphoreType.DMA,
          pltpu.SemaphoreType.DMA,
          pltpu.SemaphoreType.DMA,
          pltpu.SemaphoreType.DMA,
          pltpu.SemaphoreType.DMA,
          pltpu.SemaphoreType.DMA,
          pltpu.SemaphoreType.DMA,
      ],
  )
  def agg(x_hbm, src_hbm, dst_hbm, acc_out, cnt_out,
          rows0_v, rows1_v, sx0, sx1, sx2, sx3, dx0, dx1, dx2, dx3,
          stail_v, dtail_v, ones_c, ones_t, zo_v,
          acc_sh, cnt_sh, sg0, sg1, si0, si1, si2, si3, ss0, ss1):
    cid = lax.axis_index("c")
    tid = lax.axis_index("s")
    wid = cid * NS + tid
    ebase = wid * EPW
    zeros16 = jnp.zeros((16,), jnp.float32)
    ones16 = jnp.ones((16,), jnp.float32)
    rows = (rows0_v, rows1_v)
    sidx = (sx0, sx1, sx2, sx3)
    didx = (dx0, dx1, dx2, dx3)
    sgs = (sg0, sg1)
    sis = (si0, si1, si2, si3)
    sss = (ss0, ss1)

    def zrow(r, carry):
      for g in range(D // 16):
        rows0_v[r, pl.ds(g * 16, 16)] = zeros16
      return carry
    lax.fori_loop(0, ZR, zrow, 0)

    def zcnt(k, carry):
      zo_v[pl.ds(k * 16, 16)] = zeros16
      return carry
    lax.fori_loop(0, CB // 16, zcnt, 0)
    zo_v[pl.ds(CB - 16, 16)] = zeros16

    for j in range(RPT // ZR):
      pltpu.make_async_copy(rows0_v.at[pl.ds(0, ZR)],
                            acc_sh.at[pl.ds(tid * RPT + j * ZR, ZR)], ss0).start()

    @pl.when(tid == 0)
    def _():
      pltpu.make_async_copy(rows0_v.at[pl.ds(0, TAIL)],
                            acc_sh.at[pl.ds(NS * RPT, TAIL)], ss1).start()

    @pl.when(tid < N // CNT_B)
    def _():
      pltpu.make_async_copy(zo_v.at[pl.ds(0, CNT_B)],
                            cnt_sh.at[pl.ds(tid * CNT_B, CNT_B)], ss1).start()

    def fones(k, carry):
      ones_c[pl.ds(k * 16, 16)] = ones16
      return carry
    lax.fori_loop(0, C // 16, fones, 0)

    def ftones(k, carry):
      ones_t[pl.ds(k * 16, 16)] = ones16
      return carry
    lax.fori_loop(0, ETAIL // 16, ftones, 0)

    for j in range(RPT // ZR):
      pltpu.make_async_copy(rows0_v.at[pl.ds(0, ZR)],
                            acc_sh.at[pl.ds(tid * RPT + j * ZR, ZR)], ss0).wait()

    @pl.when(tid == 0)
    def _():
      pltpu.make_async_copy(rows0_v.at[pl.ds(0, TAIL)],
                            acc_sh.at[pl.ds(NS * RPT, TAIL)], ss1).wait()

    @pl.when(tid < N // CNT_B)
    def _():
      pltpu.make_async_copy(zo_v.at[pl.ds(0, CNT_B)],
                            cnt_sh.at[pl.ds(tid * CNT_B, CNT_B)], ss1).wait()

    plsc.subcore_barrier()

    def idx_start(g, q):
      base = pl.multiple_of(ebase + g * C, 8)
      pltpu.make_async_copy(src_hbm.at[pl.ds(base, C)], sidx[q], sis[q]).start()
      pltpu.make_async_copy(dst_hbm.at[pl.ds(base, C)], didx[q], sis[q]).start()

    def idx_wait(q):
      pltpu.make_async_copy(src_hbm.at[pl.ds(0, C)], sidx[q], sis[q]).wait()
      pltpu.make_async_copy(dst_hbm.at[pl.ds(0, C)], didx[q], sis[q]).wait()

    def gather_start(q, b):
      pltpu.make_async_copy(x_hbm.at[sidx[q]], rows[b], sgs[b]).start()

    def gather_wait(q, b):
      pltpu.make_async_copy(x_hbm.at[sidx[q]], rows[b], sgs[b]).wait()

    def scat_start(q, b):
      pltpu.make_async_copy(rows[b], acc_sh.at[didx[q]], sss[b]).start(add=True)
      pltpu.make_async_copy(ones_c, cnt_sh.at[didx[q]], sss[b]).start(add=True)

    def scat_wait(q, b):
      pltpu.make_async_copy(rows[b], acc_sh.at[didx[q]], sss[b]).wait()
      pltpu.make_async_copy(ones_c, cnt_sh.at[didx[q]], sss[b]).wait()

    idx_start(0, 0)
    idx_start(1, 1)
    idx_wait(0)
    gather_start(0, 0)
    idx_wait(1)
    gather_start(1, 1)
    idx_start(2, 2)
    idx_start(3, 3)

    def quad(k4, carry):
      g0 = 4 * k4
      for j in range(4):
        g = g0 + j
        q = j
        b = j % 2
        gather_wait(q, b)
        scat_start(q, b)
        idx_wait((q + 2) % 4)
        scat_wait(q, b)
        gather_start((q + 2) % 4, b)
        idx_start(g + 4, q)
      return carry
    NQ = (CHUNKS - 8) // 4 + 1
    lax.fori_loop(0, NQ, quad, 0)

    for j in range(4):
      g = NQ * 4 + j
      q = j
      b = j % 2
      gather_wait(q, b)
      scat_start(q, b)
      if j < 2:
        idx_wait((q + 2) % 4)
        scat_wait(q, b)
        gather_start((q + 2) % 4, b)
      else:
        scat_wait(q, b)

    tbase = pl.multiple_of(ebase + CHUNKS * C, 8)
    pltpu.sync_copy(src_hbm.at[pl.ds(tbase, ETAIL)], stail_v)
    pltpu.sync_copy(dst_hbm.at[pl.ds(tbase, ETAIL)], dtail_v)
    pltpu.make_async_copy(x_hbm.at[stail_v], rows0_v.at[pl.ds(0, ETAIL)], sg0).start()
    pltpu.make_async_copy(x_hbm.at[stail_v], rows0_v.at[pl.ds(0, ETAIL)], sg0).wait()
    pltpu.sync_copy(rows0_v.at[pl.ds(0, ETAIL)], acc_sh.at[dtail_v], add=True)
    pltpu.sync_copy(ones_t, cnt_sh.at[dtail_v], add=True)

    plsc.subcore_barrier()

    for j in range(RPT // ZR):
      b = j % 2
      r0 = tid * RPT + j * ZR
      if j >= 2:
        r_prev = tid * RPT + (j - 2) * ZR
        pltpu.make_async_copy(rows[b].at[pl.ds(0, ZR)],
                              acc_out.at[cid, pl.ds(r_prev, ZR)], sgs[b]).wait()
      pltpu.sync_copy(acc_sh.at[pl.ds(r0, ZR)], rows[b].at[pl.ds(0, ZR)])
      pltpu.make_async_copy(rows[b].at[pl.ds(0, ZR)],
                            acc_out.at[cid, pl.ds(r0, ZR)], sgs[b]).start()
    for j in (RPT // ZR - 2, RPT // ZR - 1):
      b = j % 2
      r0 = tid * RPT + j * ZR
      pltpu.make_async_copy(rows[b].at[pl.ds(0, ZR)],
                            acc_out.at[cid, pl.ds(r0, ZR)], sgs[b]).wait()

    @pl.when(tid == 0)
    def _():
      pltpu.sync_copy(acc_sh.at[pl.ds(NS * RPT, TAIL)], rows0_v.at[pl.ds(0, TAIL)])
      pltpu.sync_copy(rows0_v.at[pl.ds(0, TAIL)],
                      acc_out.at[cid, pl.ds(NS * RPT, TAIL)])

    @pl.when(tid < N // CNT_B)
    def _():
      pltpu.sync_copy(cnt_sh.at[pl.ds(tid * CNT_B, CNT_B)], zo_v.at[pl.ds(0, CNT_B)])
      pltpu.sync_copy(zo_v.at[pl.ds(0, CNT_B)],
                      cnt_out.at[pl.ds(cid * N + tid * CNT_B, CNT_B)])

  parts, cnt_flat = agg(x, src, dst)
  return parts, cnt_flat


def _tc_finish(parts, cnt_flat, x, wlt, b2, wrt):
  BN = 2000

  def body(parts_ref, cnts_ref, x_ref, wl_ref, b_ref, wr_ref, o_ref):
    sums = parts_ref[0] + parts_ref[1]
    cnt = cnts_ref[0] + cnts_ref[1]
    mean = sums * (1.0 / jnp.maximum(cnt, 1.0))
    dn = (((1,), (1,)), ((), ()))
    out = (lax.dot_general(mean, wl_ref[...], dn,
                           preferred_element_type=jnp.float32,
                           precision=lax.Precision.HIGHEST)
           + lax.dot_general(x_ref[...], wr_ref[...], dn,
                             preferred_element_type=jnp.float32,
                             precision=lax.Precision.HIGHEST)
           + b_ref[...])
    nrm = jnp.sqrt(jnp.sum(out * out, axis=1, keepdims=True))
    o_ref[...] = out / jnp.maximum(nrm, 1e-12)

  return pl.pallas_call(
      body,
      grid=(N // BN,),
      in_specs=[
          pl.BlockSpec((NC, BN, D), lambda i: (0, i, 0)),
          pl.BlockSpec((NC, BN, 1), lambda i: (0, i, 0)),
          pl.BlockSpec((BN, D), lambda i: (i, 0)),
          pl.BlockSpec((D, D), lambda i: (0, 0)),
          pl.BlockSpec((1, D), lambda i: (0, 0)),
          pl.BlockSpec((D, D), lambda i: (0, 0)),
      ],
      out_specs=pl.BlockSpec((BN, D), lambda i: (i, 0)),
      out_shape=jax.ShapeDtypeStruct((N, D), jnp.float32),
  )(parts, cnt_flat, x, wlt, b2, wrt)


def kernel(x, edge_index, attr, W_l, b_l, W_r):
  src = edge_index[0]
  dst = edge_index[1]
  parts, cnts = _sc_aggregate(x, src, dst)
  out = _tc_finish(parts, cnts.reshape(NC, N, 1), x,
                   W_l, b_l.reshape(1, D), W_r)
  return (out, edge_index, attr)


# speedup vs baseline: 13.5475x; 1.0068x over previous
import functools

import jax
import jax.numpy as jnp
from jax import lax
from jax.experimental import pallas as pl
from jax.experimental.pallas import tpu as pltpu
from jax.experimental.pallas import tpu_sc as plsc

N = 10000
E = 320000
D = 128
NC = 2
NS = 16
NW = NC * NS
EPW = E // NW
C = 176
CHUNKS = 56
ETAIL = EPW - CHUNKS * C
CB = 1000
RPT = 624
ZR = 104
TAIL = N - NS * RPT
CNT_B = 1000


def _sc_aggregate(x, src, dst):
  mesh = plsc.VectorSubcoreMesh(core_axis_name="c", subcore_axis_name="s")

  @functools.partial(
      pl.kernel,
      out_type=(
          jax.ShapeDtypeStruct((NC, N, D), jnp.float32),
          jax.ShapeDtypeStruct((NC * N,), jnp.float32),
      ),
      mesh=mesh,
      scratch_types=[
          pltpu.VMEM((C, D), jnp.float32),
          pltpu.VMEM((C, D), jnp.float32),
          pltpu.VMEM((C,), jnp.int32),
          pltpu.VMEM((C,), jnp.int32),
          pltpu.VMEM((C,), jnp.int32),
          pltpu.VMEM((C,), jnp.int32),
          pltpu.VMEM((C,), jnp.int32),
          pltpu.VMEM((C,), jnp.int32),
          pltpu.VMEM((C,), jnp.int32),
          pltpu.VMEM((C,), jnp.int32),
          pltpu.VMEM((ETAIL,), jnp.int32),
          pltpu.VMEM((ETAIL,), jnp.int32),
          pltpu.VMEM((C,), jnp.float32),
          pltpu.VMEM((ETAIL,), jnp.float32),
          pltpu.VMEM((CB,), jnp.float32),
          pltpu.VMEM_SHARED((N, D), jnp.float32),
          pltpu.VMEM_SHARED((N,), jnp.float32),
          pltpu.SemaphoreType.DMA,
          pltpu.SemaphoreType.DMA,
          pltpu.SemaphoreType.DMA,
          pltpu.SemaphoreType.DMA,
          pltpu.SemaphoreType.DMA,
          pltpu.SemaphoreType.DMA,
          pltpu.SemaphoreType.DMA,
          pltpu.SemaphoreType.DMA,
      ],
  )
  def agg(x_hbm, src_hbm, dst_hbm, acc_out, cnt_out,
          rows0_v, rows1_v, sx0, sx1, sx2, sx3, dx0, dx1, dx2, dx3,
          stail_v, dtail_v, ones_c, ones_t, zo_v,
          acc_sh, cnt_sh, sg0, sg1, si0, si1, si2, si3, ss0, ss1):
    cid = lax.axis_index("c")
    tid = lax.axis_index("s")
    wid = cid * NS + tid
    ebase = wid * EPW
    zeros16 = jnp.zeros((16,), jnp.float32)
    ones16 = jnp.ones((16,), jnp.float32)
    rows = (rows0_v, rows1_v)
    sidx = (sx0, sx1, sx2, sx3)
    didx = (dx0, dx1, dx2, dx3)
    sgs = (sg0, sg1)
    sis = (si0, si1, si2, si3)
    sss = (ss0, ss1)

    def zrow(r, carry):
      for g in range(D // 16):
        rows0_v[r, pl.ds(g * 16, 16)] = zeros16
      return carry
    lax.fori_loop(0, ZR, zrow, 0)

    def zcnt(k, carry):
      zo_v[pl.ds(k * 16, 16)] = zeros16
      return carry
    lax.fori_loop(0, CB // 16, zcnt, 0)
    zo_v[pl.ds(CB - 16, 16)] = zeros16

    for j in range(RPT // ZR):
      pltpu.make_async_copy(rows0_v.at[pl.ds(0, ZR)],
                            acc_sh.at[pl.ds(tid * RPT + j * ZR, ZR)], ss0).start()

    @pl.when(tid == 0)
    def _():
      pltpu.make_async_copy(rows0_v.at[pl.ds(0, TAIL)],
                            acc_sh.at[pl.ds(NS * RPT, TAIL)], ss1).start()

    @pl.when(tid < N // CNT_B)
    def _():
      pltpu.make_async_copy(zo_v.at[pl.ds(0, CNT_B)],
                            cnt_sh.at[pl.ds(tid * CNT_B, CNT_B)], ss1).start()

    def fones(k, carry):
      ones_c[pl.ds(k * 16, 16)] = ones16
      return carry
    lax.fori_loop(0, C // 16, fones, 0)

    def ftones(k, carry):
      ones_t[pl.ds(k * 16, 16)] = ones16
      return carry
    lax.fori_loop(0, ETAIL // 16, ftones, 0)

    for j in range(RPT // ZR):
      pltpu.make_async_copy(rows0_v.at[pl.ds(0, ZR)],
                            acc_sh.at[pl.ds(tid * RPT + j * ZR, ZR)], ss0).wait()

    @pl.when(tid == 0)
    def _():
      pltpu.make_async_copy(rows0_v.at[pl.ds(0, TAIL)],
                            acc_sh.at[pl.ds(NS * RPT, TAIL)], ss1).wait()

    @pl.when(tid < N // CNT_B)
    def _():
      pltpu.make_async_copy(zo_v.at[pl.ds(0, CNT_B)],
                            cnt_sh.at[pl.ds(tid * CNT_B, CNT_B)], ss1).wait()

    plsc.subcore_barrier()

    def idx_start(g, q):
      base = pl.multiple_of(ebase + g * C, 8)
      pltpu.make_async_copy(src_hbm.at[pl.ds(base, C)], sidx[q], sis[q]).start()
      pltpu.make_async_copy(dst_hbm.at[pl.ds(base, C)], didx[q], sis[q]).start()

    def idx_wait(q):
      pltpu.make_async_copy(src_hbm.at[pl.ds(0, C)], sidx[q], sis[q]).wait()
      pltpu.make_async_copy(dst_hbm.at[pl.ds(0, C)], didx[q], sis[q]).wait()

    def gather_start(q, b):
      pltpu.make_async_copy(x_hbm.at[sidx[q]], rows[b], sgs[b]).start()

    def gather_wait(q, b):
      pltpu.make_async_copy(x_hbm.at[sidx[q]], rows[b], sgs[b]).wait()

    def scat_start(q, b):
      pltpu.make_async_copy(rows[b], acc_sh.at[didx[q]], sss[b]).start(add=True)
      pltpu.make_async_copy(ones_c, cnt_sh.at[didx[q]], sss[b]).start(add=True)

    def scat_wait(q, b):
      pltpu.make_async_copy(rows[b], acc_sh.at[didx[q]], sss[b]).wait()
      pltpu.make_async_copy(ones_c, cnt_sh.at[didx[q]], sss[b]).wait()

    idx_start(0, 0)
    idx_start(1, 1)
    idx_wait(0)
    gather_start(0, 0)
    idx_wait(1)
    gather_start(1, 1)
    idx_start(2, 2)
    idx_start(3, 3)

    def quad(k4, carry):
      g0 = 4 * k4
      for j in range(4):
        g = g0 + j
        q = j
        b = j % 2
        gather_wait(q, b)
        scat_start(q, b)
        idx_wait((q + 2) % 4)
        scat_wait(q, b)
        gather_start((q + 2) % 4, b)
        idx_start(g + 4, q)
      return carry
    NQ = (CHUNKS - 8) // 4 + 1
    lax.fori_loop(0, NQ, quad, 0)

    for j in range(4):
      g = NQ * 4 + j
      q = j
      b = j % 2
      gather_wait(q, b)
      scat_start(q, b)
      if j < 2:
        idx_wait((q + 2) % 4)
        scat_wait(q, b)
        gather_start((q + 2) % 4, b)
      else:
        scat_wait(q, b)

    tbase = pl.multiple_of(ebase + CHUNKS * C, 8)
    pltpu.sync_copy(src_hbm.at[pl.ds(tbase, ETAIL)], stail_v)
    pltpu.sync_copy(dst_hbm.at[pl.ds(tbase, ETAIL)], dtail_v)
    pltpu.make_async_copy(x_hbm.at[stail_v], rows0_v.at[pl.ds(0, ETAIL)], sg0).start()
    pltpu.make_async_copy(x_hbm.at[stail_v], rows0_v.at[pl.ds(0, ETAIL)], sg0).wait()
    pltpu.sync_copy(rows0_v.at[pl.ds(0, ETAIL)], acc_sh.at[dtail_v], add=True)
    pltpu.sync_copy(ones_t, cnt_sh.at[dtail_v], add=True)

    plsc.subcore_barrier()

    for j in range(RPT // ZR):
      b = j % 2
      r0 = tid * RPT + j * ZR
      if j >= 2:
        r_prev = tid * RPT + (j - 2) * ZR
        pltpu.make_async_copy(rows[b].at[pl.ds(0, ZR)],
                              acc_out.at[cid, pl.ds(r_prev, ZR)], sgs[b]).wait()
      pltpu.sync_copy(acc_sh.at[pl.ds(r0, ZR)], rows[b].at[pl.ds(0, ZR)])
      pltpu.make_async_copy(rows[b].at[pl.ds(0, ZR)],
                            acc_out.at[cid, pl.ds(r0, ZR)], sgs[b]).start()
    for j in (RPT // ZR - 2, RPT // ZR - 1):
      b = j % 2
      r0 = tid * RPT + j * ZR
      pltpu.make_async_copy(rows[b].at[pl.ds(0, ZR)],
                            acc_out.at[cid, pl.ds(r0, ZR)], sgs[b]).wait()

    @pl.when(tid == 0)
    def _():
      pltpu.sync_copy(acc_sh.at[pl.ds(NS * RPT, TAIL)], rows0_v.at[pl.ds(0, TAIL)])
      pltpu.sync_copy(rows0_v.at[pl.ds(0, TAIL)],
                      acc_out.at[cid, pl.ds(NS * RPT, TAIL)])

    @pl.when(tid < N // CNT_B)
    def _():
      pltpu.sync_copy(cnt_sh.at[pl.ds(tid * CNT_B, CNT_B)], zo_v.at[pl.ds(0, CNT_B)])
      pltpu.sync_copy(zo_v.at[pl.ds(0, CNT_B)],
                      cnt_out.at[pl.ds(cid * N + tid * CNT_B, CNT_B)])

  parts, cnt_flat = agg(x, src, dst)
  return parts, cnt_flat


_DN = (((1,), (1,)), ((), ()))


def _tc_selfpart(x, wr, b2):
  BN = 2000

  def body(x_ref, wr_ref, b_ref, o_ref):
    o_ref[...] = lax.dot_general(
        x_ref[...], wr_ref[...], _DN, preferred_element_type=jnp.float32,
        precision=lax.Precision.HIGHEST) + b_ref[...]

  return pl.pallas_call(
      body,
      grid=(N // BN,),
      in_specs=[
          pl.BlockSpec((BN, D), lambda i: (i, 0)),
          pl.BlockSpec((D, D), lambda i: (0, 0)),
          pl.BlockSpec((1, D), lambda i: (0, 0)),
      ],
      out_specs=pl.BlockSpec((BN, D), lambda i: (i, 0)),
      out_shape=jax.ShapeDtypeStruct((N, D), jnp.float32),
  )(x, wr, b2)


def _tc_finish(parts, cnt3, sp, wl):
  BN = 2000

  def body(parts_ref, cnts_ref, sp_ref, wl_ref, o_ref):
    sums = parts_ref[0] + parts_ref[1]
    cnt = cnts_ref[0] + cnts_ref[1]
    mean = sums * (1.0 / jnp.maximum(cnt, 1.0))
    out = lax.dot_general(mean, wl_ref[...], _DN,
                          preferred_element_type=jnp.float32,
                          precision=lax.Precision.HIGHEST) + sp_ref[...]
    nrm = jnp.sqrt(jnp.sum(out * out, axis=1, keepdims=True))
    o_ref[...] = out / jnp.maximum(nrm, 1e-12)

  return pl.pallas_call(
      body,
      grid=(N // BN,),
      in_specs=[
          pl.BlockSpec((NC, BN, D), lambda i: (0, i, 0)),
          pl.BlockSpec((NC, BN, 1), lambda i: (0, i, 0)),
          pl.BlockSpec((BN, D), lambda i: (i, 0)),
          pl.BlockSpec((D, D), lambda i: (0, 0)),
      ],
      out_specs=pl.BlockSpec((BN, D), lambda i: (i, 0)),
      out_shape=jax.ShapeDtypeStruct((N, D), jnp.float32),
  )(parts, cnt3, sp, wl)


def kernel(x, edge_index, attr, W_l, b_l, W_r):
  src = edge_index[0]
  dst = edge_index[1]
  sp = _tc_selfpart(x, W_r, b_l.reshape(1, D))
  parts, cnts = _sc_aggregate(x, src, dst)
  out = _tc_finish(parts, cnts.reshape(NC, N, 1), sp, W_l)
  return (out, edge_index, attr)
